# baseline (speedup 1.0000x reference)
import jax
import jax.numpy as jnp
from jax import lax
from jax.experimental import pallas as pl
from jax.experimental.pallas import tpu as pltpu
from jax.experimental.pallas import tpu_sc as plsc

VOCAB = 36
POS = 7
ED = 32
OD = 128
B = 16384
RS = 40

NC = 2
NS = 16
NW = NC * NS
SPW = B // NW


def _table_body(emb_ref, w_ref, b_ref, t_ref):
    emb = emb_ref[...]
    for p in range(POS):
        tp = jnp.dot(emb, w_ref[p], preferred_element_type=jnp.float32)
        if p == POS - 1:
            tp = tp + b_ref[...]
        t_ref[pl.ds(p * RS, VOCAB), :] = tp


def _build_table(emb, w3, b2):
    return pl.pallas_call(
        _table_body,
        out_shape=jax.ShapeDtypeStruct((POS * RS, OD), jnp.float32),
    )(emb, w3, b2)


def _sc_body(t_hbm, x_hbm, out_hbm, tv, tvb, xv, bv, ov):
    wid = lax.axis_index("s") * NC + lax.axis_index("c")
    base = wid * SPW
    pltpu.sync_copy(t_hbm, tv)
    pltpu.sync_copy(x_hbm.at[pl.ds(base * POS, SPW * POS)],
                    xv.at[pl.ds(0, SPW * POS)])

    lanes = lax.iota(jnp.int32, 16)

    @plsc.parallel_loop(0, POS * RS * OD // 32, 1, unroll=4)
    def packt(i):
        r = i >> 2
        j = i & 3
        lo = tv[pl.ds(r * OD + j * 16, 16)]
        hi = tv[pl.ds(r * OD + 64 + j * 16, 16)]
        tvb[pl.ds(r * OD + j * 32, 32)] = plsc.pack(
            lo, hi, format=plsc.PackFormat.INTERLEAVED)

    @plsc.parallel_loop(0, SPW * POS // 16, 1, unroll=4)
    def addr(i):
        f = lanes + i * 16
        q = (f * 9363) >> 16
        p = f - q * POS
        a = xv[pl.ds(i * 16, 16)] * OD + p * (RS * OD)
        plsc.store_scatter(bv, [q * 8 + p], a)

    @plsc.parallel_loop(0, SPW // 2, 1, unroll=4)
    def pair(s2):
        av = bv[pl.ds(s2 * 16, 16)]
        for h in range(2):
            addrs = [av[h * 8 + p] for p in range(POS)]
            obase = s2 * (2 * OD) + h * OD
            for j in range(OD // 32):
                acc = tvb[pl.ds(addrs[0] + j * 32, 32)]
                for p in range(1, POS):
                    acc = acc + tvb[pl.ds(addrs[p] + j * 32, 32)]
                lo, hi = plsc.unpack(acc, format=plsc.PackFormat.INTERLEAVED)
                ov[pl.ds(obase + j * 16, 16)] = lo
                ov[pl.ds(obase + 64 + j * 16, 16)] = hi

    pltpu.sync_copy(ov, out_hbm.at[pl.ds(base * OD, SPW * OD)])


_sc_call = pl.kernel(
    _sc_body,
    mesh=plsc.VectorSubcoreMesh(core_axis_name="c", subcore_axis_name="s"),
    compiler_params=pltpu.CompilerParams(needs_layout_passes=False),
    out_type=jax.ShapeDtypeStruct((B * OD,), jnp.float32),
    scratch_types=[
        pltpu.VMEM((POS * RS * OD,), jnp.float32),
        pltpu.VMEM((POS * RS * OD,), jnp.bfloat16),
        pltpu.VMEM((SPW * POS + 16,), jnp.int32),
        pltpu.VMEM((SPW * 8,), jnp.int32),
        pltpu.VMEM((SPW * OD,), jnp.float32),
    ],
)


def kernel(x, emb, W, b):
    t = _build_table(emb, W.reshape(POS, ED, OD), b.reshape(1, OD))
    out_flat = _sc_call(t.reshape(-1), x.reshape(-1).astype(jnp.int32))
    return out_flat.reshape(B, OD)


# speedup vs baseline: 1.1524x; 1.1524x over previous
import jax
import jax.numpy as jnp
from jax import lax
from jax.experimental import pallas as pl
from jax.experimental.pallas import tpu as pltpu
from jax.experimental.pallas import tpu_sc as plsc

VOCAB = 36
POS = 7
ED = 32
OD = 128
B = 16384
RS = 40

NC = 2
NS = 16
NW = NC * NS
SPW = B // NW


def _table_body(emb_ref, w_ref, b_ref, t_ref):
    emb = emb_ref[...]
    for p in range(POS):
        tp = jnp.dot(emb, w_ref[p], preferred_element_type=jnp.float32)
        if p == POS - 1:
            tp = tp + b_ref[...]
        t_ref[pl.ds(p * RS, VOCAB), :] = tp


def _build_table(emb, w3, b2):
    return pl.pallas_call(
        _table_body,
        out_shape=jax.ShapeDtypeStruct((POS * RS, OD), jnp.float32),
    )(emb, w3, b2)


def _sc_body(t_hbm, x_hbm, out_hbm, tv, tvb, xv, bv, ov):
    wid = lax.axis_index("s") * NC + lax.axis_index("c")
    base = wid * SPW
    pltpu.sync_copy(t_hbm, tv)
    pltpu.sync_copy(x_hbm.at[pl.ds(base * POS, SPW * POS)],
                    xv.at[pl.ds(0, SPW * POS)])

    lanes = lax.iota(jnp.int32, 16)

    @plsc.parallel_loop(0, POS * RS * OD // 32, 1, unroll=4)
    def packt(i):
        r = i >> 2
        j = i & 3
        lo = tv[pl.ds(r * OD + j * 16, 16)]
        hi = tv[pl.ds(r * OD + 64 + j * 16, 16)]
        packed = plsc.pack(lo, hi, format=plsc.PackFormat.INTERLEAVED)
        tvb[pl.ds(r * (OD // 2) + j * 16, 16)] = plsc.bitcast(packed, jnp.int32)

    @plsc.parallel_loop(0, SPW * POS // 16, 1, unroll=4)
    def addr(i):
        f = lanes + i * 16
        q = (f * 9363) >> 16
        p = f - q * POS
        a = xv[pl.ds(i * 16, 16)] * (OD // 2) + p * (RS * OD // 2)
        plsc.store_scatter(bv, [q * 8 + p], a)

    @plsc.parallel_loop(0, SPW // 2, 1, unroll=4)
    def pair(s2):
        av = bv[pl.ds(s2 * 16, 16)]
        for h in range(2):
            addrs = [av[h * 8 + p] for p in range(POS)]
            obase = s2 * (2 * OD) + h * OD
            for j in range(OD // 32):
                acc = plsc.bitcast(tvb[pl.ds(addrs[0] + j * 16, 16)],
                                   jnp.bfloat16)
                for p in range(1, POS):
                    acc = acc + plsc.bitcast(
                        tvb[pl.ds(addrs[p] + j * 16, 16)], jnp.bfloat16)
                lo, hi = plsc.unpack(acc, format=plsc.PackFormat.INTERLEAVED)
                ov[pl.ds(obase + j * 16, 16)] = lo
                ov[pl.ds(obase + 64 + j * 16, 16)] = hi

    pltpu.sync_copy(ov, out_hbm.at[pl.ds(base * OD, SPW * OD)])


_sc_call = pl.kernel(
    _sc_body,
    mesh=plsc.VectorSubcoreMesh(core_axis_name="c", subcore_axis_name="s"),
    compiler_params=pltpu.CompilerParams(needs_layout_passes=False),
    out_type=jax.ShapeDtypeStruct((B * OD,), jnp.float32),
    scratch_types=[
        pltpu.VMEM((POS * RS * OD,), jnp.float32),
        pltpu.VMEM((POS * RS * OD // 2,), jnp.int32),
        pltpu.VMEM((SPW * POS + 16,), jnp.int32),
        pltpu.VMEM((SPW * 8,), jnp.int32),
        pltpu.VMEM((SPW * OD,), jnp.float32),
    ],
)


def kernel(x, emb, W, b):
    t = _build_table(emb, W.reshape(POS, ED, OD), b.reshape(1, OD))
    out_flat = _sc_call(t.reshape(-1), x.reshape(-1).astype(jnp.int32))
    return out_flat.reshape(B, OD)


# speedup vs baseline: 1.2542x; 1.0884x over previous
import jax
import jax.numpy as jnp
from jax import lax
from jax.experimental import pallas as pl
from jax.experimental.pallas import tpu as pltpu
from jax.experimental.pallas import tpu_sc as plsc

VOCAB = 36
POS = 7
ED = 32
OD = 128
B = 16384
RW = VOCAB * (OD // 2) // OD

NC = 2
NS = 16
NW = NC * NS
SPW = B // NW


def _table_body(emb_ref, w_ref, b_ref, t_ref):
    emb = emb_ref[...]
    for p in range(POS):
        tp = jnp.dot(emb, w_ref[p], preferred_element_type=jnp.float32)
        if p == POS - 1:
            tp = tp + b_ref[...]
        lo = lax.bitcast_convert_type(
            tp[:, : OD // 2].astype(jnp.bfloat16), jnp.uint16
        ).astype(jnp.uint32)
        hi = lax.bitcast_convert_type(
            tp[:, OD // 2 :].astype(jnp.bfloat16), jnp.uint16
        ).astype(jnp.uint32)
        packed = lax.bitcast_convert_type(lo | (hi << 16), jnp.int32)
        for i in range(RW):
            row = jnp.concatenate(
                [packed[2 * i : 2 * i + 1, :], packed[2 * i + 1 : 2 * i + 2, :]],
                axis=1,
            )
            t_ref[pl.ds(RW * p + i, 1), :] = row


def _build_table(emb, w3, b2):
    return pl.pallas_call(
        _table_body,
        out_shape=jax.ShapeDtypeStruct((128, OD), jnp.int32),
    )(emb, w3, b2)


def _sc_body(t_hbm, x_hbm, out_hbm, tvb, xv, bv, ov):
    wid = lax.axis_index("s") * NC + lax.axis_index("c")
    base = wid * SPW
    pltpu.sync_copy(t_hbm, tvb)
    pltpu.sync_copy(x_hbm.at[pl.ds(base * POS, SPW * POS)],
                    xv.at[pl.ds(0, SPW * POS)])

    lanes = lax.iota(jnp.int32, 16)

    @plsc.parallel_loop(0, SPW * POS // 16, 1, unroll=4)
    def addr(i):
        f = lanes + i * 16
        q = (f * 9363) >> 16
        p = f - q * POS
        a = xv[pl.ds(i * 16, 16)] * (OD // 2) + p * (VOCAB * OD // 2)
        plsc.store_scatter(bv, [q * 8 + p], a)

    @plsc.parallel_loop(0, SPW // 2, 1, unroll=4)
    def pair(s2):
        av = bv[pl.ds(s2 * 16, 16)]
        for h in range(2):
            addrs = [av[h * 8 + p] for p in range(POS)]
            obase = s2 * (2 * OD) + h * OD
            for j in range(OD // 32):
                acc = plsc.bitcast(tvb[pl.ds(addrs[0] + j * 16, 16)],
                                   jnp.bfloat16)
                for p in range(1, POS):
                    acc = acc + plsc.bitcast(
                        tvb[pl.ds(addrs[p] + j * 16, 16)], jnp.bfloat16)
                lo, hi = plsc.unpack(acc, format=plsc.PackFormat.INTERLEAVED)
                ov[pl.ds(obase + j * 16, 16)] = lo
                ov[pl.ds(obase + 64 + j * 16, 16)] = hi

    pltpu.sync_copy(ov, out_hbm.at[pl.ds(base * OD, SPW * OD)])


_sc_call = pl.kernel(
    _sc_body,
    mesh=plsc.VectorSubcoreMesh(core_axis_name="c", subcore_axis_name="s"),
    compiler_params=pltpu.CompilerParams(needs_layout_passes=False),
    out_type=jax.ShapeDtypeStruct((B * OD,), jnp.float32),
    scratch_types=[
        pltpu.VMEM((128 * OD,), jnp.int32),
        pltpu.VMEM((SPW * POS + 16,), jnp.int32),
        pltpu.VMEM((SPW * 8,), jnp.int32),
        pltpu.VMEM((SPW * OD,), jnp.float32),
    ],
)


def kernel(x, emb, W, b):
    t = _build_table(emb, W.reshape(POS, ED, OD), b.reshape(1, OD))
    out_flat = _sc_call(t.reshape(-1), x.reshape(-1).astype(jnp.int32))
    return out_flat.reshape(B, OD)
